# baseline (speedup 1.0000x reference)
import functools

import jax
import jax.numpy as jnp
from jax import lax
from jax.experimental import pallas as pl
from jax.experimental.pallas import tpu as pltpu
from jax.experimental.pallas import tpu_sc as plsc

TOPK = 8
NUM_EXPERTS = 64
BLOCK_ROWS = 512

_SC_INFO = plsc.get_sparse_core_info()
_NC, _NS = _SC_INFO.num_cores, _SC_INFO.num_subcores
_NW = _NC * _NS


def _logits_body(x_ref, wt_ref, s_ref):
    s_ref[...] = jnp.dot(x_ref[...], wt_ref[...],
                         preferred_element_type=jnp.float32)


def _tc_logits(x, wt):
    n = x.shape[0]
    return pl.pallas_call(
        _logits_body,
        grid=(n // BLOCK_ROWS,),
        in_specs=[
            pl.BlockSpec((BLOCK_ROWS, x.shape[1]), lambda i: (i, 0)),
            pl.BlockSpec((x.shape[1], NUM_EXPERTS), lambda i: (0, 0)),
        ],
        out_specs=pl.BlockSpec((BLOCK_ROWS, NUM_EXPERTS), lambda i: (i, 0)),
        out_shape=jax.ShapeDtypeStruct((n, NUM_EXPERTS), jnp.float32),
    )(x, wt)


def _lane_gather(v, idx):
    dnums = lax.GatherDimensionNumbers(
        offset_dims=(), collapsed_slice_dims=(0,), start_index_map=(0,))
    return lax.gather(v, idx[:, None], dnums, (1,),
                      mode=lax.GatherScatterMode.PROMISE_IN_BOUNDS)


def _make_sc_router(n_tokens):
    tpw = n_tokens // _NW
    pairs = tpw // 2
    mesh = plsc.VectorSubcoreMesh(core_axis_name="c", subcore_axis_name="s")

    @functools.partial(
        pl.kernel,
        out_type=[
            jax.ShapeDtypeStruct((n_tokens * TOPK,), jnp.float32),
            jax.ShapeDtypeStruct((n_tokens * TOPK,), jnp.int32),
        ],
        mesh=mesh,
        compiler_params=pltpu.CompilerParams(needs_layout_passes=False),
        scratch_types=[
            pltpu.VMEM((tpw * NUM_EXPERTS,), jnp.float32),
            pltpu.VMEM((tpw * TOPK,), jnp.float32),
            pltpu.VMEM((tpw * TOPK,), jnp.int32),
        ],
    )
    def router(scores_hbm, w_hbm, i_hbm, sc_v, wout_v, iout_v):
        wid = lax.axis_index("s") * _NC + lax.axis_index("c")
        base = wid * tpw
        pltpu.sync_copy(scores_hbm.at[pl.ds(base * NUM_EXPERTS,
                                            tpw * NUM_EXPERTS)], sc_v)

        lanes = lax.iota(jnp.int32, 16)
        low8 = lanes < 8
        idx_group = [lanes + 16 * g for g in range(4)]
        rot8 = (lanes + 8) & 15

        def rev(v):
            return lax.rev(v, (0,))

        def merge(a, b):
            mk = jnp.where(low8, a[0], rev(b[0]))
            mv = jnp.where(low8, a[1], rev(b[1]))
            return plsc.sort_key_val(mk, mv, descending=True)

        def top8(off):
            srt = [
                plsc.sort_key_val(sc_v[pl.ds(off + 16 * g, 16)],
                                  idx_group[g], descending=True)
                for g in range(4)
            ]
            k, v = merge(merge(srt[0], srt[1]), merge(srt[2], srt[3]))
            s = 1.0 / (1.0 + jnp.exp(-k))
            tot = jnp.sum(jnp.where(low8, s, 0.0))
            return s / tot, v

        def body(p, carry):
            we, ie = top8(p * 2 * NUM_EXPERTS)
            wo, io = top8(p * 2 * NUM_EXPERTS + NUM_EXPERTS)
            pw = jnp.where(low8, we, _lane_gather(wo, rot8))
            pi = jnp.where(low8, ie, _lane_gather(io, rot8))
            wout_v[pl.ds(p * 16, 16)] = pw
            iout_v[pl.ds(p * 16, 16)] = pi
            return carry

        lax.fori_loop(0, pairs, body, 0)
        pltpu.sync_copy(wout_v, w_hbm.at[pl.ds(base * TOPK, tpw * TOPK)])
        pltpu.sync_copy(iout_v, i_hbm.at[pl.ds(base * TOPK, tpw * TOPK)])

    return router


def _fused_body(x_ref, w_ref, w_out_ref, i_out_ref):
    scores = lax.dot_general(w_ref[...], x_ref[...],
                             (((1,), (1,)), ((), ())),
                             preferred_element_type=jnp.float32)
    rows = scores.shape[1]
    colf = lax.broadcasted_iota(
        jnp.int32, (NUM_EXPERTS, rows), 0).astype(jnp.float32)

    vals = []
    idxs = []
    for _ in range(TOPK):
        m = jnp.max(scores, axis=0, keepdims=True)
        amax = jnp.min(jnp.where(scores == m, colf, float(NUM_EXPERTS)),
                       axis=0, keepdims=True)
        vals.append(m)
        idxs.append(amax)
        scores = jnp.where(colf == amax, -jnp.inf, scores)

    top_vals = jnp.concatenate(vals, axis=0)
    top_idxf = jnp.concatenate(idxs, axis=0)
    s = 1.0 / (1.0 + jnp.exp(-top_vals))
    w = s / jnp.sum(s, axis=0, keepdims=True)
    w_out_ref[...] = w.T
    i_out_ref[...] = top_idxf.T.astype(jnp.int32)


@jax.jit
def kernel(x, weight):
    n = x.shape[0]
    grid = (n // BLOCK_ROWS,)
    w_out, i_out = pl.pallas_call(
        _fused_body,
        grid=grid,
        in_specs=[
            pl.BlockSpec((BLOCK_ROWS, x.shape[1]), lambda i: (i, 0)),
            pl.BlockSpec((NUM_EXPERTS, x.shape[1]), lambda i: (0, 0)),
        ],
        out_specs=[
            pl.BlockSpec((BLOCK_ROWS, TOPK), lambda i: (i, 0)),
            pl.BlockSpec((BLOCK_ROWS, TOPK), lambda i: (i, 0)),
        ],
        out_shape=[
            jax.ShapeDtypeStruct((n, TOPK), jnp.float32),
            jax.ShapeDtypeStruct((n, TOPK), jnp.int32),
        ],
    )(x, weight)
    return (w_out, i_out)


# speedup vs baseline: 1.1727x; 1.1727x over previous
import functools

import jax
import jax.numpy as jnp
from jax import lax
from jax.experimental import pallas as pl
from jax.experimental.pallas import tpu as pltpu
from jax.experimental.pallas import tpu_sc as plsc

TOPK = 8
NUM_EXPERTS = 64
BLOCK_ROWS = 1024

_SC_INFO = plsc.get_sparse_core_info()
_NC, _NS = _SC_INFO.num_cores, _SC_INFO.num_subcores
_NW = _NC * _NS


def _logits_body(x_ref, wt_ref, s_ref):
    s_ref[...] = jnp.dot(x_ref[...], wt_ref[...],
                         preferred_element_type=jnp.float32)


def _tc_logits(x, wt):
    n = x.shape[0]
    return pl.pallas_call(
        _logits_body,
        grid=(n // BLOCK_ROWS,),
        in_specs=[
            pl.BlockSpec((BLOCK_ROWS, x.shape[1]), lambda i: (i, 0)),
            pl.BlockSpec((x.shape[1], NUM_EXPERTS), lambda i: (0, 0)),
        ],
        out_specs=pl.BlockSpec((BLOCK_ROWS, NUM_EXPERTS), lambda i: (i, 0)),
        out_shape=jax.ShapeDtypeStruct((n, NUM_EXPERTS), jnp.float32),
    )(x, wt)


def _lane_gather(v, idx):
    dnums = lax.GatherDimensionNumbers(
        offset_dims=(), collapsed_slice_dims=(0,), start_index_map=(0,))
    return lax.gather(v, idx[:, None], dnums, (1,),
                      mode=lax.GatherScatterMode.PROMISE_IN_BOUNDS)


def _make_sc_router(n_tokens):
    tpw = n_tokens // _NW
    pairs = tpw // 2
    mesh = plsc.VectorSubcoreMesh(core_axis_name="c", subcore_axis_name="s")

    @functools.partial(
        pl.kernel,
        out_type=[
            jax.ShapeDtypeStruct((n_tokens * TOPK,), jnp.float32),
            jax.ShapeDtypeStruct((n_tokens * TOPK,), jnp.int32),
        ],
        mesh=mesh,
        compiler_params=pltpu.CompilerParams(needs_layout_passes=False),
        scratch_types=[
            pltpu.VMEM((tpw * NUM_EXPERTS,), jnp.float32),
            pltpu.VMEM((tpw * TOPK,), jnp.float32),
            pltpu.VMEM((tpw * TOPK,), jnp.int32),
        ],
    )
    def router(scores_hbm, w_hbm, i_hbm, sc_v, wout_v, iout_v):
        wid = lax.axis_index("s") * _NC + lax.axis_index("c")
        base = wid * tpw
        pltpu.sync_copy(scores_hbm.at[pl.ds(base * NUM_EXPERTS,
                                            tpw * NUM_EXPERTS)], sc_v)

        lanes = lax.iota(jnp.int32, 16)
        low8 = lanes < 8
        idx_group = [lanes + 16 * g for g in range(4)]
        rot8 = (lanes + 8) & 15

        def rev(v):
            return lax.rev(v, (0,))

        def merge(a, b):
            mk = jnp.where(low8, a[0], rev(b[0]))
            mv = jnp.where(low8, a[1], rev(b[1]))
            return plsc.sort_key_val(mk, mv, descending=True)

        def top8(off):
            srt = [
                plsc.sort_key_val(sc_v[pl.ds(off + 16 * g, 16)],
                                  idx_group[g], descending=True)
                for g in range(4)
            ]
            k, v = merge(merge(srt[0], srt[1]), merge(srt[2], srt[3]))
            s = 1.0 / (1.0 + jnp.exp(-k))
            tot = jnp.sum(jnp.where(low8, s, 0.0))
            return s / tot, v

        def body(p, carry):
            we, ie = top8(p * 2 * NUM_EXPERTS)
            wo, io = top8(p * 2 * NUM_EXPERTS + NUM_EXPERTS)
            pw = jnp.where(low8, we, _lane_gather(wo, rot8))
            pi = jnp.where(low8, ie, _lane_gather(io, rot8))
            wout_v[pl.ds(p * 16, 16)] = pw
            iout_v[pl.ds(p * 16, 16)] = pi
            return carry

        lax.fori_loop(0, pairs, body, 0)
        pltpu.sync_copy(wout_v, w_hbm.at[pl.ds(base * TOPK, tpw * TOPK)])
        pltpu.sync_copy(iout_v, i_hbm.at[pl.ds(base * TOPK, tpw * TOPK)])

    return router


def _fused_body(x_ref, w_ref, w_out_ref, i_out_ref):
    scores = lax.dot_general(w_ref[...], x_ref[...],
                             (((1,), (1,)), ((), ())),
                             preferred_element_type=jnp.float32)
    rows = scores.shape[1]
    colf = lax.broadcasted_iota(
        jnp.int32, (NUM_EXPERTS, rows), 0).astype(jnp.float32)

    vals = []
    idxs = []
    for _ in range(TOPK):
        m = jnp.max(scores, axis=0, keepdims=True)
        amax = jnp.min(jnp.where(scores == m, colf, float(NUM_EXPERTS)),
                       axis=0, keepdims=True)
        vals.append(m)
        idxs.append(amax)
        scores = jnp.where(colf == amax, -jnp.inf, scores)

    top_vals = jnp.concatenate(vals, axis=0)
    top_idxf = jnp.concatenate(idxs, axis=0)
    s = 1.0 / (1.0 + jnp.exp(-top_vals))
    w = s / jnp.sum(s, axis=0, keepdims=True)
    w_out_ref[...] = w.T
    i_out_ref[...] = top_idxf.T.astype(jnp.int32)


@jax.jit
def kernel(x, weight):
    n = x.shape[0]
    grid = (n // BLOCK_ROWS,)
    w_out, i_out = pl.pallas_call(
        _fused_body,
        grid=grid,
        in_specs=[
            pl.BlockSpec((BLOCK_ROWS, x.shape[1]), lambda i: (i, 0)),
            pl.BlockSpec((NUM_EXPERTS, x.shape[1]), lambda i: (0, 0)),
        ],
        out_specs=[
            pl.BlockSpec((BLOCK_ROWS, TOPK), lambda i: (i, 0)),
            pl.BlockSpec((BLOCK_ROWS, TOPK), lambda i: (i, 0)),
        ],
        out_shape=[
            jax.ShapeDtypeStruct((n, TOPK), jnp.float32),
            jax.ShapeDtypeStruct((n, TOPK), jnp.int32),
        ],
    )(x, weight)
    return (w_out, i_out)


# speedup vs baseline: 1.2173x; 1.0380x over previous
import functools

import jax
import jax.numpy as jnp
from jax import lax
from jax.experimental import pallas as pl
from jax.experimental.pallas import tpu as pltpu
from jax.experimental.pallas import tpu_sc as plsc

TOPK = 8
NUM_EXPERTS = 64
BLOCK_ROWS = 2048

_SC_INFO = plsc.get_sparse_core_info()
_NC, _NS = _SC_INFO.num_cores, _SC_INFO.num_subcores
_NW = _NC * _NS


def _logits_body(x_ref, wt_ref, s_ref):
    s_ref[...] = jnp.dot(x_ref[...], wt_ref[...],
                         preferred_element_type=jnp.float32)


def _tc_logits(x, wt):
    n = x.shape[0]
    return pl.pallas_call(
        _logits_body,
        grid=(n // BLOCK_ROWS,),
        in_specs=[
            pl.BlockSpec((BLOCK_ROWS, x.shape[1]), lambda i: (i, 0)),
            pl.BlockSpec((x.shape[1], NUM_EXPERTS), lambda i: (0, 0)),
        ],
        out_specs=pl.BlockSpec((BLOCK_ROWS, NUM_EXPERTS), lambda i: (i, 0)),
        out_shape=jax.ShapeDtypeStruct((n, NUM_EXPERTS), jnp.float32),
    )(x, wt)


def _lane_gather(v, idx):
    dnums = lax.GatherDimensionNumbers(
        offset_dims=(), collapsed_slice_dims=(0,), start_index_map=(0,))
    return lax.gather(v, idx[:, None], dnums, (1,),
                      mode=lax.GatherScatterMode.PROMISE_IN_BOUNDS)


def _make_sc_router(n_tokens):
    tpw = n_tokens // _NW
    pairs = tpw // 2
    mesh = plsc.VectorSubcoreMesh(core_axis_name="c", subcore_axis_name="s")

    @functools.partial(
        pl.kernel,
        out_type=[
            jax.ShapeDtypeStruct((n_tokens * TOPK,), jnp.float32),
            jax.ShapeDtypeStruct((n_tokens * TOPK,), jnp.int32),
        ],
        mesh=mesh,
        compiler_params=pltpu.CompilerParams(needs_layout_passes=False),
        scratch_types=[
            pltpu.VMEM((tpw * NUM_EXPERTS,), jnp.float32),
            pltpu.VMEM((tpw * TOPK,), jnp.float32),
            pltpu.VMEM((tpw * TOPK,), jnp.int32),
        ],
    )
    def router(scores_hbm, w_hbm, i_hbm, sc_v, wout_v, iout_v):
        wid = lax.axis_index("s") * _NC + lax.axis_index("c")
        base = wid * tpw
        pltpu.sync_copy(scores_hbm.at[pl.ds(base * NUM_EXPERTS,
                                            tpw * NUM_EXPERTS)], sc_v)

        lanes = lax.iota(jnp.int32, 16)
        low8 = lanes < 8
        idx_group = [lanes + 16 * g for g in range(4)]
        rot8 = (lanes + 8) & 15

        def rev(v):
            return lax.rev(v, (0,))

        def merge(a, b):
            mk = jnp.where(low8, a[0], rev(b[0]))
            mv = jnp.where(low8, a[1], rev(b[1]))
            return plsc.sort_key_val(mk, mv, descending=True)

        def top8(off):
            srt = [
                plsc.sort_key_val(sc_v[pl.ds(off + 16 * g, 16)],
                                  idx_group[g], descending=True)
                for g in range(4)
            ]
            k, v = merge(merge(srt[0], srt[1]), merge(srt[2], srt[3]))
            s = 1.0 / (1.0 + jnp.exp(-k))
            tot = jnp.sum(jnp.where(low8, s, 0.0))
            return s / tot, v

        def body(p, carry):
            we, ie = top8(p * 2 * NUM_EXPERTS)
            wo, io = top8(p * 2 * NUM_EXPERTS + NUM_EXPERTS)
            pw = jnp.where(low8, we, _lane_gather(wo, rot8))
            pi = jnp.where(low8, ie, _lane_gather(io, rot8))
            wout_v[pl.ds(p * 16, 16)] = pw
            iout_v[pl.ds(p * 16, 16)] = pi
            return carry

        lax.fori_loop(0, pairs, body, 0)
        pltpu.sync_copy(wout_v, w_hbm.at[pl.ds(base * TOPK, tpw * TOPK)])
        pltpu.sync_copy(iout_v, i_hbm.at[pl.ds(base * TOPK, tpw * TOPK)])

    return router


def _fused_body(x_ref, w_ref, w_out_ref, i_out_ref):
    scores = lax.dot_general(w_ref[...], x_ref[...],
                             (((1,), (1,)), ((), ())),
                             preferred_element_type=jnp.float32)
    rows = scores.shape[1]
    colf = lax.broadcasted_iota(
        jnp.int32, (NUM_EXPERTS, rows), 0).astype(jnp.float32)

    vals = []
    idxs = []
    for _ in range(TOPK):
        m = jnp.max(scores, axis=0, keepdims=True)
        amax = jnp.min(jnp.where(scores == m, colf, float(NUM_EXPERTS)),
                       axis=0, keepdims=True)
        vals.append(m)
        idxs.append(amax)
        scores = jnp.where(colf == amax, -jnp.inf, scores)

    top_vals = jnp.concatenate(vals, axis=0)
    top_idxf = jnp.concatenate(idxs, axis=0)
    s = 1.0 / (1.0 + jnp.exp(-top_vals))
    w = s / jnp.sum(s, axis=0, keepdims=True)
    w_out_ref[...] = w.T
    i_out_ref[...] = top_idxf.T.astype(jnp.int32)


@jax.jit
def kernel(x, weight):
    n = x.shape[0]
    grid = (n // BLOCK_ROWS,)
    w_out, i_out = pl.pallas_call(
        _fused_body,
        grid=grid,
        in_specs=[
            pl.BlockSpec((BLOCK_ROWS, x.shape[1]), lambda i: (i, 0)),
            pl.BlockSpec((NUM_EXPERTS, x.shape[1]), lambda i: (0, 0)),
        ],
        out_specs=[
            pl.BlockSpec((BLOCK_ROWS, TOPK), lambda i: (i, 0)),
            pl.BlockSpec((BLOCK_ROWS, TOPK), lambda i: (i, 0)),
        ],
        out_shape=[
            jax.ShapeDtypeStruct((n, TOPK), jnp.float32),
            jax.ShapeDtypeStruct((n, TOPK), jnp.int32),
        ],
    )(x, weight)
    return (w_out, i_out)


# speedup vs baseline: 1.2233x; 1.0050x over previous
import functools

import jax
import jax.numpy as jnp
from jax import lax
from jax.experimental import pallas as pl
from jax.experimental.pallas import tpu as pltpu
from jax.experimental.pallas import tpu_sc as plsc

TOPK = 8
NUM_EXPERTS = 64
BLOCK_ROWS = 2048

_SC_INFO = plsc.get_sparse_core_info()
_NC, _NS = _SC_INFO.num_cores, _SC_INFO.num_subcores
_NW = _NC * _NS


def _logits_body(x_ref, wt_ref, s_ref):
    s_ref[...] = jnp.dot(x_ref[...], wt_ref[...],
                         preferred_element_type=jnp.float32)


def _tc_logits(x, wt):
    n = x.shape[0]
    return pl.pallas_call(
        _logits_body,
        grid=(n // BLOCK_ROWS,),
        in_specs=[
            pl.BlockSpec((BLOCK_ROWS, x.shape[1]), lambda i: (i, 0)),
            pl.BlockSpec((x.shape[1], NUM_EXPERTS), lambda i: (0, 0)),
        ],
        out_specs=pl.BlockSpec((BLOCK_ROWS, NUM_EXPERTS), lambda i: (i, 0)),
        out_shape=jax.ShapeDtypeStruct((n, NUM_EXPERTS), jnp.float32),
    )(x, wt)


def _lane_gather(v, idx):
    dnums = lax.GatherDimensionNumbers(
        offset_dims=(), collapsed_slice_dims=(0,), start_index_map=(0,))
    return lax.gather(v, idx[:, None], dnums, (1,),
                      mode=lax.GatherScatterMode.PROMISE_IN_BOUNDS)


def _make_sc_router(n_tokens):
    tpw = n_tokens // _NW
    pairs = tpw // 2
    mesh = plsc.VectorSubcoreMesh(core_axis_name="c", subcore_axis_name="s")

    @functools.partial(
        pl.kernel,
        out_type=[
            jax.ShapeDtypeStruct((n_tokens * TOPK,), jnp.float32),
            jax.ShapeDtypeStruct((n_tokens * TOPK,), jnp.int32),
        ],
        mesh=mesh,
        compiler_params=pltpu.CompilerParams(needs_layout_passes=False),
        scratch_types=[
            pltpu.VMEM((tpw * NUM_EXPERTS,), jnp.float32),
            pltpu.VMEM((tpw * TOPK,), jnp.float32),
            pltpu.VMEM((tpw * TOPK,), jnp.int32),
        ],
    )
    def router(scores_hbm, w_hbm, i_hbm, sc_v, wout_v, iout_v):
        wid = lax.axis_index("s") * _NC + lax.axis_index("c")
        base = wid * tpw
        pltpu.sync_copy(scores_hbm.at[pl.ds(base * NUM_EXPERTS,
                                            tpw * NUM_EXPERTS)], sc_v)

        lanes = lax.iota(jnp.int32, 16)
        low8 = lanes < 8
        idx_group = [lanes + 16 * g for g in range(4)]
        rot8 = (lanes + 8) & 15

        def rev(v):
            return lax.rev(v, (0,))

        def merge(a, b):
            mk = jnp.where(low8, a[0], rev(b[0]))
            mv = jnp.where(low8, a[1], rev(b[1]))
            return plsc.sort_key_val(mk, mv, descending=True)

        def top8(off):
            srt = [
                plsc.sort_key_val(sc_v[pl.ds(off + 16 * g, 16)],
                                  idx_group[g], descending=True)
                for g in range(4)
            ]
            k, v = merge(merge(srt[0], srt[1]), merge(srt[2], srt[3]))
            s = 1.0 / (1.0 + jnp.exp(-k))
            tot = jnp.sum(jnp.where(low8, s, 0.0))
            return s / tot, v

        def body(p, carry):
            we, ie = top8(p * 2 * NUM_EXPERTS)
            wo, io = top8(p * 2 * NUM_EXPERTS + NUM_EXPERTS)
            pw = jnp.where(low8, we, _lane_gather(wo, rot8))
            pi = jnp.where(low8, ie, _lane_gather(io, rot8))
            wout_v[pl.ds(p * 16, 16)] = pw
            iout_v[pl.ds(p * 16, 16)] = pi
            return carry

        lax.fori_loop(0, pairs, body, 0)
        pltpu.sync_copy(wout_v, w_hbm.at[pl.ds(base * TOPK, tpw * TOPK)])
        pltpu.sync_copy(iout_v, i_hbm.at[pl.ds(base * TOPK, tpw * TOPK)])

    return router


def _fused_body(x_ref, w_ref, w_out_ref, i_out_ref):
    scores = lax.dot_general(w_ref[...], x_ref[...],
                             (((1,), (1,)), ((), ())),
                             preferred_element_type=jnp.float32)
    rows = scores.shape[1]
    colf = lax.broadcasted_iota(
        jnp.int32, (NUM_EXPERTS, rows), 0).astype(jnp.float32)

    sig = jax.nn.sigmoid(scores)

    vals = []
    idxs = []
    for _ in range(TOPK):
        m = jnp.max(sig, axis=0, keepdims=True)
        amax = jnp.min(jnp.where(sig == m, colf, float(NUM_EXPERTS)),
                       axis=0, keepdims=True)
        vals.append(m)
        idxs.append(amax)
        sig = jnp.where(colf == amax, -1.0, sig)

    top_vals = jnp.concatenate(vals, axis=0)
    top_idxf = jnp.concatenate(idxs, axis=0)
    w = top_vals / jnp.sum(top_vals, axis=0, keepdims=True)
    w_out_ref[...] = w.T
    i_out_ref[...] = top_idxf.T.astype(jnp.int32)


@jax.jit
def kernel(x, weight):
    n = x.shape[0]
    grid = (n // BLOCK_ROWS,)
    w_out, i_out = pl.pallas_call(
        _fused_body,
        grid=grid,
        in_specs=[
            pl.BlockSpec((BLOCK_ROWS, x.shape[1]), lambda i: (i, 0)),
            pl.BlockSpec((NUM_EXPERTS, x.shape[1]), lambda i: (0, 0)),
        ],
        out_specs=[
            pl.BlockSpec((BLOCK_ROWS, TOPK), lambda i: (i, 0)),
            pl.BlockSpec((BLOCK_ROWS, TOPK), lambda i: (i, 0)),
        ],
        out_shape=[
            jax.ShapeDtypeStruct((n, TOPK), jnp.float32),
            jax.ShapeDtypeStruct((n, TOPK), jnp.int32),
        ],
    )(x, weight)
    return (w_out, i_out)
